# baseline (speedup 1.0000x reference)
import functools

import jax
import jax.numpy as jnp
from jax import lax
from jax.experimental import pallas as pl
from jax.experimental.pallas import tpu as pltpu
from jax.experimental.pallas import tpu_sc as plsc

B = 16384
F = 8
V = 12
D = 64
D2 = 2 * D
ROWS = B * F
TAB = F * V

NC = 2
NS = 16
CHUNK = 128
LANES = 16
NBUF = 2
LA = 1

NCH0 = 48
NCH1 = 16
ROWS0 = NCH0 * CHUNK
ROWS1 = NCH1 * CHUNK
assert NS * (ROWS0 + ROWS1) == ROWS


def _sc_gather_body(x_hbm, tab_hbm, out_hbm, idx_v, tab_sp, *rest):
    bufs = rest[:NBUF]
    gsems = rest[NBUF:2 * NBUF]
    psems = rest[2 * NBUF:3 * NBUF]

    cid = lax.axis_index("c")
    sid = lax.axis_index("s")

    @pl.when(sid == 0)
    def _stage_table():
        stage = bufs[0].at[pl.ds(0, TAB)]
        pltpu.sync_copy(tab_hbm, stage)
        pltpu.sync_copy(stage, tab_sp)

    def stage_idx(base, nrows):
        pltpu.sync_copy(x_hbm.at[pl.ds(base, nrows)], idx_v.at[pl.ds(0, nrows)])
        off = (lax.iota(jnp.int32, 16) % F) * V
        for o in range(nrows // LANES):
            sl = pl.ds(o * LANES, LANES)
            idx_v[sl] = idx_v[sl] + off

    def ring(base, nchunk):
        gat = [None] * NBUF
        put = [None] * NBUF
        for t in range(nchunk + LA):
            if t < nchunk:
                bi = t % NBUF
                if put[bi] is not None:
                    put[bi].wait()
                gat[bi] = pltpu.async_copy(
                    tab_sp.at[idx_v.at[pl.ds(t * CHUNK, CHUNK)]],
                    bufs[bi], gsems[bi])
            if t >= LA:
                c = t - LA
                pb = c % NBUF
                gat[pb].wait()
                put[pb] = pltpu.async_copy(
                    bufs[pb], out_hbm.at[pl.ds(base + c * CHUNK, CHUNK)],
                    psems[pb])
        for p in put:
            if p is not None:
                p.wait()

    @pl.when(cid == 0)
    def _idx0():
        stage_idx(sid * ROWS0, ROWS0)

    @pl.when(cid == 1)
    def _idx1():
        stage_idx(NS * ROWS0 + sid * ROWS1, ROWS1)

    plsc.subcore_barrier()

    @pl.when(cid == 0)
    def _run0():
        ring(sid * ROWS0, NCH0)

    @pl.when(cid == 1)
    def _run1():
        ring(NS * ROWS0 + sid * ROWS1, NCH1)


_sc_gather = functools.partial(
    pl.kernel,
    out_type=jax.ShapeDtypeStruct((ROWS, D2), jnp.float32),
    mesh=plsc.VectorSubcoreMesh(core_axis_name="c", subcore_axis_name="s"),
    scratch_types=(
        [pltpu.VMEM((ROWS0,), jnp.int32),
         pltpu.VMEM_SHARED((TAB, D2), jnp.float32)]
        + [pltpu.VMEM((CHUNK, D2), jnp.float32) for _ in range(NBUF)]
        + [pltpu.SemaphoreType.DMA for _ in range(2 * NBUF)]
    ),
)(_sc_gather_body)


def kernel(x, mean_table, std_table):
    tab = jnp.concatenate(
        [mean_table.reshape(TAB, D), std_table.reshape(TAB, D)], axis=1)
    x1 = x.reshape(ROWS).astype(jnp.int32)
    out = _sc_gather(x1, tab)
    return out.reshape(B, F, D2)


# speedup vs baseline: 1.0035x; 1.0035x over previous
import functools

import jax
import jax.numpy as jnp
from jax import lax
from jax.experimental import pallas as pl
from jax.experimental.pallas import tpu as pltpu
from jax.experimental.pallas import tpu_sc as plsc

B = 16384
F = 8
V = 12
D = 64
D2 = 2 * D
ROWS = B * F
TAB = F * V

NC = 2
NS = 16
CHUNK = 128
LANES = 16
NBUF = 2
LA = 1

NCH0 = 16
NCH1 = 48
ROWS0 = NCH0 * CHUNK
ROWS1 = NCH1 * CHUNK
assert NS * (ROWS0 + ROWS1) == ROWS


def _sc_gather_body(x_hbm, tab_hbm, out_hbm, idx_v, tab_sp, *rest):
    bufs = rest[:NBUF]
    gsems = rest[NBUF:2 * NBUF]
    psems = rest[2 * NBUF:3 * NBUF]

    cid = lax.axis_index("c")
    sid = lax.axis_index("s")

    @pl.when(sid == 0)
    def _stage_table():
        stage = bufs[0].at[pl.ds(0, TAB)]
        pltpu.sync_copy(tab_hbm, stage)
        pltpu.sync_copy(stage, tab_sp)

    def stage_idx(base, nrows):
        pltpu.sync_copy(x_hbm.at[pl.ds(base, nrows)], idx_v.at[pl.ds(0, nrows)])
        off = (lax.iota(jnp.int32, 16) % F) * V
        for o in range(nrows // LANES):
            sl = pl.ds(o * LANES, LANES)
            idx_v[sl] = idx_v[sl] + off

    def ring(base, nchunk):
        gat = [None] * NBUF
        put = [None] * NBUF
        for t in range(nchunk + LA):
            if t < nchunk:
                bi = t % NBUF
                if put[bi] is not None:
                    put[bi].wait()
                gat[bi] = pltpu.async_copy(
                    tab_sp.at[idx_v.at[pl.ds(t * CHUNK, CHUNK)]],
                    bufs[bi], gsems[bi])
            if t >= LA:
                c = t - LA
                pb = c % NBUF
                gat[pb].wait()
                put[pb] = pltpu.async_copy(
                    bufs[pb], out_hbm.at[pl.ds(base + c * CHUNK, CHUNK)],
                    psems[pb])
        for p in put:
            if p is not None:
                p.wait()

    @pl.when(cid == 0)
    def _idx0():
        stage_idx(sid * ROWS0, ROWS0)

    @pl.when(cid == 1)
    def _idx1():
        stage_idx(NS * ROWS0 + sid * ROWS1, ROWS1)

    plsc.subcore_barrier()

    @pl.when(cid == 0)
    def _run0():
        ring(sid * ROWS0, NCH0)

    @pl.when(cid == 1)
    def _run1():
        ring(NS * ROWS0 + sid * ROWS1, NCH1)


_sc_gather = functools.partial(
    pl.kernel,
    out_type=jax.ShapeDtypeStruct((ROWS, D2), jnp.float32),
    mesh=plsc.VectorSubcoreMesh(core_axis_name="c", subcore_axis_name="s"),
    scratch_types=(
        [pltpu.VMEM((max(ROWS0, ROWS1),), jnp.int32),
         pltpu.VMEM_SHARED((TAB, D2), jnp.float32)]
        + [pltpu.VMEM((CHUNK, D2), jnp.float32) for _ in range(NBUF)]
        + [pltpu.SemaphoreType.DMA for _ in range(2 * NBUF)]
    ),
)(_sc_gather_body)


def kernel(x, mean_table, std_table):
    tab = jnp.concatenate(
        [mean_table.reshape(TAB, D), std_table.reshape(TAB, D)], axis=1)
    x1 = x.reshape(ROWS).astype(jnp.int32)
    out = _sc_gather(x1, tab)
    return out.reshape(B, F, D2)


# speedup vs baseline: 1.2353x; 1.2310x over previous
import functools

import jax
import jax.numpy as jnp
from jax import lax
from jax.experimental import pallas as pl
from jax.experimental.pallas import tpu as pltpu
from jax.experimental.pallas import tpu_sc as plsc

B = 16384
F = 8
V = 12
D = 64
D2 = 2 * D
ROWS = B * F
TAB = F * V

NC = 2
NS = 16
NW = NC * NS
PER_W = ROWS // NW
CHUNK = 128
NCHUNK = PER_W // CHUNK
LANES = 16
NBUF = 6
LA = 3


def _sc_gather_body(x_hbm, tab_hbm, out_hbm, idx_v, tab_stage, tab_sp, *rest):
    bufs = rest[:NBUF]
    gsems = rest[NBUF:2 * NBUF]
    psems = rest[2 * NBUF:3 * NBUF]

    sid = lax.axis_index("s")
    wid = sid * NC + lax.axis_index("c")
    base = wid * PER_W

    @pl.when(sid == 0)
    def _stage_table():
        pltpu.sync_copy(tab_hbm, tab_stage)
        pltpu.sync_copy(tab_stage, tab_sp)

    pltpu.sync_copy(x_hbm.at[wid], idx_v)
    off = (lax.iota(jnp.int32, 16) % F) * V

    plsc.subcore_barrier()

    gat = [None] * NBUF
    put = [None] * NBUF
    for t in range(NCHUNK + LA):
        if t < NCHUNK:
            bi = t % NBUF
            if put[bi] is not None:
                put[bi].wait()
            for o in range(CHUNK // LANES):
                sl = pl.ds(o * LANES, LANES)
                idx_v[t, sl] = idx_v[t, sl] + off
            gat[bi] = pltpu.async_copy(
                tab_sp.at[idx_v.at[t]], bufs[bi], gsems[bi])
        if t >= LA:
            c = t - LA
            pb = c % NBUF
            gat[pb].wait()
            put[pb] = pltpu.async_copy(
                bufs[pb], out_hbm.at[pl.ds(base + c * CHUNK, CHUNK)],
                psems[pb])
    for p in put:
        if p is not None:
            p.wait()


_sc_gather = functools.partial(
    pl.kernel,
    out_type=jax.ShapeDtypeStruct((ROWS, D2), jnp.float32),
    mesh=plsc.VectorSubcoreMesh(core_axis_name="c", subcore_axis_name="s"),
    scratch_types=(
        [pltpu.VMEM((NCHUNK, CHUNK), jnp.int32),
         pltpu.VMEM((TAB, D2), jnp.float32),
         pltpu.VMEM_SHARED((TAB, D2), jnp.float32)]
        + [pltpu.VMEM((CHUNK, D2), jnp.float32) for _ in range(NBUF)]
        + [pltpu.SemaphoreType.DMA for _ in range(2 * NBUF)]
    ),
)(_sc_gather_body)


def kernel(x, mean_table, std_table):
    tab = jnp.concatenate(
        [mean_table.reshape(TAB, D), std_table.reshape(TAB, D)], axis=1)
    x3 = x.reshape(NW, NCHUNK, CHUNK).astype(jnp.int32)
    out = _sc_gather(x3, tab)
    return out.reshape(B, F, D2)
